# baseline (speedup 1.0000x reference)
import functools

import jax
import jax.numpy as jnp
from jax import lax
from jax.experimental import pallas as pl
from jax.experimental.pallas import tpu as pltpu
from jax.experimental.pallas import tpu_sc as plsc

B = 128
L = 512
D = 512
N = B * L
NC, NS = 2, 16
NW = NC * NS
TPW = N // NW
C = 64
NCHUNK = TPW // C
LANES = 16
G = D // LANES


def _positional_encoding(max_len, d_model):
    even_i = jnp.arange(0, d_model, 2).astype(jnp.float32)
    denominator = jnp.power(10000.0, even_i / d_model)
    position = jnp.arange(max_len, dtype=jnp.float32).reshape(max_len, 1)
    even_pe = jnp.sin(position / denominator)
    odd_pe = jnp.cos(position / denominator)
    return jnp.stack([even_pe, odd_pe], axis=2).reshape(max_len, d_model)


@functools.partial(
    pl.kernel,
    mesh=plsc.VectorSubcoreMesh(core_axis_name="c", subcore_axis_name="s"),
    out_type=jax.ShapeDtypeStruct((N, D), jnp.float32),
    scratch_types=[
        pltpu.VMEM((TPW,), jnp.int32),
        pltpu.VMEM((C, D), jnp.float32),
        pltpu.VMEM((C, D), jnp.float32),
        pltpu.SemaphoreType.DMA,
    ],
)
def _emb_pe(table_hbm, idx_hbm, pe_hbm, out_hbm, idx_v, rows_v, pe_v, sem):
    wid = lax.axis_index("s") * NC + lax.axis_index("c")
    tok_base = wid * TPW
    pltpu.sync_copy(idx_hbm.at[pl.ds(tok_base, TPW)], idx_v)

    def chunk_body(c, carry):
        pltpu.async_copy(
            table_hbm.at[idx_v.at[pl.ds(c * C, C)]], rows_v, sem
        ).wait()
        l0 = (c % (L // C)) * C
        pltpu.sync_copy(pe_hbm.at[pl.ds(l0, C)], pe_v)

        def add_body(i, acc):
            t = i // G
            g = i % G
            sl = pl.ds(g * LANES, LANES)
            plsc.addupdate(rows_v.at[t, sl], pe_v[t, sl])
            return acc

        lax.fori_loop(0, C * G, add_body, 0)

        pltpu.sync_copy(rows_v, out_hbm.at[pl.ds(tok_base + c * C, C)])
        return carry

    lax.fori_loop(0, NCHUNK, chunk_body, 0)


def kernel(x, table, start_token, end_token):
    del start_token, end_token
    pe = _positional_encoding(L, D)
    out = _emb_pe(table, x.reshape(N), pe)
    return out.reshape(B, L, D)


# speedup vs baseline: 1.5435x; 1.5435x over previous
import functools

import jax
import jax.numpy as jnp
from jax import lax
from jax.experimental import pallas as pl
from jax.experimental.pallas import tpu as pltpu
from jax.experimental.pallas import tpu_sc as plsc

B = 128
L = 512
D = 512
N = B * L
NC, NS = 2, 16
NW = NC * NS
TPW = N // NW
C = 32
NCHUNK = TPW // C
NBODY = NCHUNK // 2
LANES = 16
G = D // LANES


def _positional_encoding(max_len, d_model):
    even_i = jnp.arange(0, d_model, 2).astype(jnp.float32)
    denominator = jnp.power(10000.0, even_i / d_model)
    position = jnp.arange(max_len, dtype=jnp.float32).reshape(max_len, 1)
    even_pe = jnp.sin(position / denominator)
    odd_pe = jnp.cos(position / denominator)
    return jnp.stack([even_pe, odd_pe], axis=2).reshape(max_len, d_model)


@functools.partial(
    pl.kernel,
    mesh=plsc.VectorSubcoreMesh(core_axis_name="c", subcore_axis_name="s"),
    out_type=jax.ShapeDtypeStruct((N, D), jnp.float32),
    scratch_types=[
        pltpu.VMEM((TPW,), jnp.int32),
        pltpu.VMEM((C, D), jnp.float32),
        pltpu.VMEM((C, D), jnp.float32),
        pltpu.VMEM((C, D), jnp.float32),
        pltpu.VMEM((C, D), jnp.float32),
        pltpu.SemaphoreType.DMA,
        pltpu.SemaphoreType.DMA,
        pltpu.SemaphoreType.DMA,
        pltpu.SemaphoreType.DMA,
        pltpu.SemaphoreType.DMA,
        pltpu.SemaphoreType.DMA,
    ],
)
def _emb_pe(table_hbm, idx_hbm, pe_hbm, out_hbm,
            idx_v, rows0, rows1, pe0, pe1,
            sg0, sg1, sp0, sp1, so0, so1):
    wid = lax.axis_index("s") * NC + lax.axis_index("c")
    tok_base = wid * TPW
    pltpu.sync_copy(idx_hbm.at[pl.ds(tok_base, TPW)], idx_v)

    rows = (rows0, rows1)
    pes = (pe0, pe1)
    sg = (sg0, sg1)
    sp = (sp0, sp1)
    so = (so0, so1)

    def issue_in(c, p):
        g = pltpu.async_copy(
            table_hbm.at[idx_v.at[pl.ds(c * C, C)]], rows[p], sg[p]
        )
        l0 = (c % (L // C)) * C
        q = pltpu.async_copy(pe_hbm.at[pl.ds(l0, C)], pes[p], sp[p])
        return g, q

    def add_rows(p):
        r, q = rows[p], pes[p]

        def row_body(t, acc):
            for g in range(G):
                sl = pl.ds(g * LANES, LANES)
                plsc.addupdate(r.at[t, sl], q[t, sl])
            return acc

        lax.fori_loop(0, C, row_body, 0)

    def issue_out(c, p):
        return pltpu.async_copy(
            rows[p], out_hbm.at[pl.ds(tok_base + c * C, C)], so[p]
        )

    def wait_out(p):
        pltpu.make_async_copy(
            rows[p], out_hbm.at[pl.ds(tok_base, C)], so[p]
        ).wait()

    def body(k, carry):
        c0 = 2 * k
        c1 = 2 * k + 1

        @pl.when(k > 0)
        def _():
            wait_out(0)
            wait_out(1)

        g0, q0 = issue_in(c0, 0)
        g1, q1 = issue_in(c1, 1)

        g0.wait()
        q0.wait()
        add_rows(0)
        issue_out(c0, 0)

        g1.wait()
        q1.wait()
        add_rows(1)
        issue_out(c1, 1)
        return carry

    lax.fori_loop(0, NBODY, body, 0)
    wait_out(0)
    wait_out(1)


def kernel(x, table, start_token, end_token):
    del start_token, end_token
    pe = _positional_encoding(L, D)
    out = _emb_pe(table, x.reshape(N), pe)
    return out.reshape(B, L, D)


# speedup vs baseline: 1.6018x; 1.0378x over previous
import functools

import jax
import jax.numpy as jnp
from jax import lax
from jax.experimental import pallas as pl
from jax.experimental.pallas import tpu as pltpu
from jax.experimental.pallas import tpu_sc as plsc

B = 128
L = 512
D = 512
N = B * L
NC, NS = 2, 16
NW = NC * NS
TPW = N // NW
C = 32
NCHUNK = TPW // C
NBODY = NCHUNK // 2
LANES = 16
G = D // LANES


def _positional_encoding(max_len, d_model):
    even_i = jnp.arange(0, d_model, 2).astype(jnp.float32)
    denominator = jnp.power(10000.0, even_i / d_model)
    position = jnp.arange(max_len, dtype=jnp.float32).reshape(max_len, 1)
    even_pe = jnp.sin(position / denominator)
    odd_pe = jnp.cos(position / denominator)
    return jnp.stack([even_pe, odd_pe], axis=2).reshape(max_len, d_model)


@functools.partial(
    pl.kernel,
    mesh=plsc.VectorSubcoreMesh(core_axis_name="c", subcore_axis_name="s"),
    out_type=jax.ShapeDtypeStruct((N, D), jnp.float32),
    scratch_types=[
        pltpu.VMEM((TPW,), jnp.int32),
        pltpu.VMEM((C, D), jnp.float32),
        pltpu.VMEM((C, D), jnp.float32),
        pltpu.VMEM((C, D), jnp.float32),
        pltpu.VMEM((C, D), jnp.float32),
        pltpu.SemaphoreType.DMA,
        pltpu.SemaphoreType.DMA,
        pltpu.SemaphoreType.DMA,
        pltpu.SemaphoreType.DMA,
        pltpu.SemaphoreType.DMA,
        pltpu.SemaphoreType.DMA,
    ],
)
def _emb_pe(table_hbm, idx_hbm, pe_hbm, out_hbm,
            idx_v, rows0, rows1, pe0, pe1,
            sg0, sg1, sp0, sp1, so0, so1):
    wid = lax.axis_index("s") * NC + lax.axis_index("c")
    tok_base = wid * TPW
    pltpu.sync_copy(idx_hbm.at[pl.ds(tok_base, TPW)], idx_v)

    rows = (rows0, rows1)
    pes = (pe0, pe1)
    sg = (sg0, sg1)
    sp = (sp0, sp1)
    so = (so0, so1)

    def issue_in(c, p):
        g = pltpu.async_copy(
            table_hbm.at[idx_v.at[pl.ds(c * C, C)]], rows[p], sg[p]
        )
        l0 = (c % (L // C)) * C
        q = pltpu.async_copy(pe_hbm.at[pl.ds(l0, C)], pes[p], sp[p])
        return g, q

    def add_rows(p):
        r, q = rows[p], pes[p]

        def row_body(t, acc):
            for g0 in range(0, G, 8):
                sls = [pl.ds((g0 + j) * LANES, LANES) for j in range(8)]
                vals = [q[t, sl] for sl in sls]
                for sl, val in zip(sls, vals):
                    plsc.addupdate(r.at[t, sl], val)
            return acc

        lax.fori_loop(0, C, row_body, 0)

    def issue_out(c, p):
        return pltpu.async_copy(
            rows[p], out_hbm.at[pl.ds(tok_base + c * C, C)], so[p]
        )

    def wait_out(p):
        pltpu.make_async_copy(
            rows[p], out_hbm.at[pl.ds(tok_base, C)], so[p]
        ).wait()

    def body(k, carry):
        c0 = 2 * k
        c1 = 2 * k + 1

        @pl.when(k > 0)
        def _():
            wait_out(0)
            wait_out(1)

        g0, q0 = issue_in(c0, 0)
        g1, q1 = issue_in(c1, 1)

        g0.wait()
        q0.wait()
        add_rows(0)
        issue_out(c0, 0)

        g1.wait()
        q1.wait()
        add_rows(1)
        issue_out(c1, 1)
        return carry

    lax.fori_loop(0, NBODY, body, 0)
    wait_out(0)
    wait_out(1)


def kernel(x, table, start_token, end_token):
    del start_token, end_token
    pe = _positional_encoding(L, D)
    out = _emb_pe(table, x.reshape(N), pe)
    return out.reshape(B, L, D)
